# baseline (speedup 1.0000x reference)
import functools

import jax
import jax.numpy as jnp
from jax import lax
from jax.experimental import pallas as pl
from jax.experimental.pallas import tpu as pltpu
from jax.experimental.pallas import tpu_sc as plsc

N_NODES = 10000
N_EDGES = 320000
D = 128

NC = 2
NS = 16
NW = NC * NS
C = 128
CPT = 80
E_PAD = NW * C * CPT
N_ACC = 10112
ROWS_TILE = N_ACC // NS


def _sc_aggregate(x, src2d, dst2d, zeros_hbm):
    mesh = plsc.VectorSubcoreMesh(core_axis_name="c", subcore_axis_name="s")

    @functools.partial(
        pl.kernel,
        out_type=jax.ShapeDtypeStruct((NC, N_ACC, D), jnp.float32),
        mesh=mesh,
        scratch_types=[
            pltpu.VMEM((CPT, C), jnp.int32),
            pltpu.VMEM((1, C), jnp.int32),
            pltpu.VMEM((1, C), jnp.int32),
            pltpu.VMEM((C, D), jnp.float32),
            pltpu.VMEM((C, D), jnp.float32),
            pltpu.VMEM_SHARED((N_ACC, D), jnp.float32),
            pltpu.SemaphoreType.DMA,
            pltpu.SemaphoreType.DMA,
            pltpu.SemaphoreType.DMA,
            pltpu.SemaphoreType.DMA,
        ],
    )
    def agg(x_hbm, src_hbm, dst_hbm, zero_hbm, out_hbm,
            src_v, dst_a, dst_b, rows_a, rows_b, acc,
            sem_ra, sem_rb, sem_da, sem_db):
        c = lax.axis_index("c")
        s = lax.axis_index("s")
        pltpu.sync_copy(zero_hbm.at[pl.ds(s * ROWS_TILE, ROWS_TILE)],
                        acc.at[pl.ds(s * ROWS_TILE, ROWS_TILE)])
        base = (c * NS + s) * CPT
        pltpu.sync_copy(src_hbm.at[pl.ds(base, CPT)], src_v)
        plsc.subcore_barrier()

        pltpu.async_copy(dst_hbm.at[base + 0], dst_a, sem_da)
        pltpu.async_copy(dst_hbm.at[base + 1], dst_b, sem_db)
        pltpu.async_copy(x_hbm.at[src_v.at[0]], rows_a, sem_ra)

        def pair(i, carry):
            a_idx = 2 * i
            b_idx = 2 * i + 1
            pltpu.async_copy(x_hbm.at[src_v.at[b_idx]], rows_b, sem_rb)
            pltpu.make_async_copy(x_hbm.at[src_v.at[a_idx]],
                                  rows_a, sem_ra).wait()
            pltpu.make_async_copy(dst_hbm.at[base + a_idx],
                                  dst_a, sem_da).wait()
            pltpu.sync_copy(rows_a, acc.at[dst_a.at[0]], add=True)

            @pl.when(a_idx + 2 < CPT)
            def _():
                pltpu.async_copy(dst_hbm.at[base + a_idx + 2],
                                 dst_a, sem_da)
                pltpu.async_copy(x_hbm.at[src_v.at[a_idx + 2]],
                                 rows_a, sem_ra)

            pltpu.make_async_copy(x_hbm.at[src_v.at[b_idx]],
                                  rows_b, sem_rb).wait()
            pltpu.make_async_copy(dst_hbm.at[base + b_idx],
                                  dst_b, sem_db).wait()
            pltpu.sync_copy(rows_b, acc.at[dst_b.at[0]], add=True)

            @pl.when(b_idx + 2 < CPT)
            def _():
                pltpu.async_copy(dst_hbm.at[base + b_idx + 2],
                                 dst_b, sem_db)

            return carry

        lax.fori_loop(0, CPT // 2, pair, 0)
        plsc.subcore_barrier()
        pltpu.sync_copy(acc.at[pl.ds(s * ROWS_TILE, ROWS_TILE)],
                        out_hbm.at[c, pl.ds(s * ROWS_TILE, ROWS_TILE)])

    return agg(x, src2d, dst2d, zeros_hbm)


def _tc_matmul(p0, p1, weight, bias2d):
    blk = 1000

    def body(p0_ref, p1_ref, w_ref, b_ref, o_ref):
        agg = p0_ref[...] + p1_ref[...]
        y = jnp.dot(agg, w_ref[...], preferred_element_type=jnp.float32)
        o_ref[...] = jnp.maximum(y + b_ref[...], 0.0)

    return pl.pallas_call(
        body,
        grid=(N_NODES // blk,),
        in_specs=[
            pl.BlockSpec((blk, D), lambda i: (i, 0)),
            pl.BlockSpec((blk, D), lambda i: (i, 0)),
            pl.BlockSpec((D, D), lambda i: (0, 0)),
            pl.BlockSpec((1, D), lambda i: (0, 0)),
        ],
        out_specs=pl.BlockSpec((blk, D), lambda i: (i, 0)),
        out_shape=jax.ShapeDtypeStruct((N_NODES, D), jnp.float32),
    )(p0, p1, weight, bias2d)


def kernel(x, edge_index, weight, bias):
    dst = edge_index[0].astype(jnp.int32)
    src = edge_index[1].astype(jnp.int32)
    pad = E_PAD - N_EDGES
    src2d = jnp.concatenate(
        [src, jnp.zeros((pad,), jnp.int32)]).reshape(NW * CPT, C)
    dst3d = jnp.concatenate(
        [dst, jnp.full((pad,), N_NODES, jnp.int32)]).reshape(NW * CPT, 1, C)
    zeros_hbm = jnp.zeros((N_ACC, D), jnp.float32)
    partials = _sc_aggregate(x, src2d, dst3d, zeros_hbm)
    return _tc_matmul(partials[0], partials[1], weight,
                      bias.reshape(1, D))


# speedup vs baseline: 1.5256x; 1.5256x over previous
import functools

import jax
import jax.numpy as jnp
from jax import lax
from jax.experimental import pallas as pl
from jax.experimental.pallas import tpu as pltpu
from jax.experimental.pallas import tpu_sc as plsc

N_NODES = 10000
N_EDGES = 320000
D = 128

NC = 2
NS = 16
C = 24
CPT = 840
E_PAD = NS * C * CPT
N_X = 10112
HALF = 5056
N_ACC = 5120
X_ROWS_TILE = N_X // NS
ACC_ROWS_TILE = N_ACC // NS
SRC_DUMMY = 10104
DST_DUMMY = HALF


def _sc_aggregate(x_pad, src3d, dstb, zeros_hbm):
    mesh = plsc.VectorSubcoreMesh(core_axis_name="c", subcore_axis_name="s")

    @functools.partial(
        pl.kernel,
        out_type=jax.ShapeDtypeStruct((NC, N_ACC, D), jnp.float32),
        mesh=mesh,
        scratch_types=[
            pltpu.VMEM_SHARED((N_X, D), jnp.float32),
            pltpu.VMEM_SHARED((N_ACC, D), jnp.float32),
            [pltpu.VMEM((1, C), jnp.int32) for _ in range(4)],
            [pltpu.VMEM((1, C), jnp.int32) for _ in range(4)],
            [pltpu.VMEM((C, D), jnp.float32) for _ in range(2)],
            [pltpu.SemaphoreType.DMA for _ in range(4)],
            [pltpu.SemaphoreType.DMA for _ in range(4)],
            [pltpu.SemaphoreType.DMA for _ in range(2)],
        ],
    )
    def agg(x_hbm, src_hbm, dst_hbm, zero_hbm, out_hbm,
            xs, acc, sv, dv, rows, sem_s, sem_d, sem_r):
        c = lax.axis_index("c")
        s = lax.axis_index("s")
        xb = s * X_ROWS_TILE
        pltpu.sync_copy(x_hbm.at[pl.ds(xb, X_ROWS_TILE)],
                        xs.at[pl.ds(xb, X_ROWS_TILE)])
        ab = s * ACC_ROWS_TILE
        pltpu.sync_copy(zero_hbm.at[pl.ds(ab, ACC_ROWS_TILE)],
                        acc.at[pl.ds(ab, ACC_ROWS_TILE)])
        plsc.subcore_barrier()

        base = s * CPT

        def load_idx(j, k):
            pltpu.async_copy(src_hbm.at[base + j], sv[k], sem_s[k])
            pltpu.async_copy(dst_hbm.at[c, base + j], dv[k], sem_d[k])

        def wait_src(j, k):
            pltpu.make_async_copy(src_hbm.at[base + j],
                                  sv[k], sem_s[k]).wait()

        def wait_dst(j, k):
            pltpu.make_async_copy(dst_hbm.at[c, base + j],
                                  dv[k], sem_d[k]).wait()

        def gather(k, r):
            pltpu.async_copy(xs.at[sv[k].at[0]], rows[r], sem_r[r])

        def wait_gather(k, r):
            pltpu.make_async_copy(xs.at[sv[k].at[0]],
                                  rows[r], sem_r[r]).wait()

        for k in range(4):
            load_idx(k, k)
        wait_src(0, 0)
        gather(0, 0)

        def quad(i, carry):
            j0 = 4 * i
            for k in range(4):
                j = j0 + k
                kn = (k + 1) % 4
                r = k % 2
                rn = (k + 1) % 2

                @pl.when(j + 1 < CPT)
                def _():
                    wait_src(j + 1, kn)
                    gather(kn, rn)

                wait_gather(k, r)
                wait_dst(j, k)
                pltpu.sync_copy(rows[r], acc.at[dv[k].at[0]], add=True)

                @pl.when(j + 4 < CPT)
                def _():
                    load_idx(j + 4, k)

            return carry

        lax.fori_loop(0, CPT // 4, quad, 0)
        plsc.subcore_barrier()
        pltpu.sync_copy(acc.at[pl.ds(ab, ACC_ROWS_TILE)],
                        out_hbm.at[c, pl.ds(ab, ACC_ROWS_TILE)])

    return agg(x_pad, src3d, dstb, zeros_hbm)


def _tc_matmul(partials, weight, bias2d):
    blk = 1264

    def body(p_ref, w_ref, b_ref, o_ref):
        y = jnp.dot(p_ref[0], w_ref[...], preferred_element_type=jnp.float32)
        o_ref[...] = jnp.maximum(y + b_ref[...], 0.0)

    return pl.pallas_call(
        body,
        grid=(N_X // blk,),
        in_specs=[
            pl.BlockSpec((1, blk, D), lambda i: (i // 4, i % 4, 0)),
            pl.BlockSpec((D, D), lambda i: (0, 0)),
            pl.BlockSpec((1, D), lambda i: (0, 0)),
        ],
        out_specs=pl.BlockSpec((blk, D), lambda i: (i, 0)),
        out_shape=jax.ShapeDtypeStruct((N_X, D), jnp.float32),
    )(partials, weight, bias2d)


def kernel(x, edge_index, weight, bias):
    dst = edge_index[0].astype(jnp.int32)
    src = edge_index[1].astype(jnp.int32)
    pad = E_PAD - N_EDGES
    src_p = jnp.concatenate([src, jnp.full((pad,), SRC_DUMMY, jnp.int32)])
    dst_p = jnp.concatenate([dst, jnp.full((pad,), N_NODES, jnp.int32)])
    src3d = src_p.reshape(NS * CPT, 1, C)
    dst0 = jnp.where(dst_p < HALF, dst_p, DST_DUMMY)
    dst1 = jnp.where(dst_p >= HALF, dst_p - HALF, DST_DUMMY)
    dstb = jnp.stack([dst0, dst1]).reshape(NC, NS * CPT, 1, C)
    x_pad = jnp.concatenate([x, jnp.zeros((N_X - N_NODES, D), jnp.float32)])
    zeros_hbm = jnp.zeros((N_ACC, D), jnp.float32)
    partials = _sc_aggregate(x_pad, src3d, dstb, zeros_hbm)
    out = _tc_matmul(partials, weight, bias.reshape(1, D))
    return out[:N_NODES]
